# baseline (speedup 1.0000x reference)
from functools import partial

import jax
import jax.numpy as jnp
from jax import lax
from jax.experimental import pallas as pl
from jax.experimental.pallas import tpu as pltpu

_R = 512
_C = 256
_K = 40
_B = 8
_NEG = float("-inf")


def _cdiv(a, b):
    return (a + b - 1) // b


def _embed_kernel(x_ref, w_ref, b_ref, cat_ref, sq_ref, *, sdim):
    cat = jnp.dot(x_ref[...], w_ref[...],
                  preferred_element_type=jnp.float32) + b_ref[...]
    cat_ref[...] = cat
    s = cat[:, :sdim]
    sq_ref[...] = jnp.sum(s * s, axis=1, keepdims=True)


def _knn_kernel(c01_ref, s_rows_ref, sq_rows_ref, out1_ref, brow_ref,
                s3_ref, sq3_ref, b3_ref, h3_ref, ht3_ref,
                wo2_ref, bo2_ref, wp1_ref, bp1_ref, wp2_ref, bp2_ref,
                idx_ref, p_ref, ssum_ref, smin_ref, smax_ref, scnt_ref,
                negbuf, *, pdim):
    t = pl.program_id(0)
    ci0 = c01_ref[t, 0]
    nc = c01_ref[t, 1]

    s_rows = s_rows_ref[...]
    sqr = sq_rows_ref[...]
    brow = brow_ref[...]
    lane = lax.broadcasted_iota(jnp.int32, (_R, _C), 1)

    def chunk_cols(cg):
        s_ct = s3_ref[cg]
        sq_c = sq3_ref[cg]
        b_c = b3_ref[cg]
        dot = lax.dot_general(s_rows, s_ct, (((1,), (0,)), ((), ())),
                              preferred_element_type=jnp.float32)
        d2 = (sqr + sq_c) - 2.0 * dot
        same = brow == b_c
        return d2, same

    def fill_body(ci, _):
        d2, same = chunk_cols(ci0 + ci)
        negbuf[ci] = jnp.where(same, -d2, _NEG)
        return 0
    lax.fori_loop(0, nc, fill_body, 0)

    big_i = jnp.int32(2 ** 30)

    def k_body(k, carry):
        topi, ntau, pj = carry

        def scan_body(ci, bc):
            bv, bj = bc
            off = (ci0 + ci) * _C
            chunk = jnp.where(lane == pj - off, _NEG, negbuf[ci])
            negbuf[ci] = chunk
            m = jnp.max(chunk, axis=1, keepdims=True)
            cand = jnp.where(chunk == m, lane, big_i)
            j = jnp.min(cand, axis=1, keepdims=True) + off
            upd = m > bv
            return jnp.where(upd, m, bv), jnp.where(upd, j, bj)

        bv0 = jnp.full((_R, 1), _NEG, jnp.float32)
        bj0 = jnp.zeros((_R, 1), jnp.int32)
        bv, bj = lax.fori_loop(0, nc, scan_body, (bv0, bj0))

        kl = lax.broadcasted_iota(jnp.int32, (1, _K), 1)
        topi = jnp.where(kl == k, bj, topi)
        ntau = jnp.where(k == _K - 1, bv, ntau)
        return topi, ntau, bj

    topi0 = jnp.zeros((_R, _K), jnp.int32)
    ntau0 = jnp.zeros((_R, 1), jnp.float32)
    pj0 = jnp.full((_R, 1), -1, jnp.int32)
    topi, ntau, _ = lax.fori_loop(0, _K, k_body, (topi0, ntau0, pj0))
    idx_ref[...] = topi
    tau = -ntau

    macc0 = jnp.zeros((_R, pdim), jnp.float32)
    xacc0 = jnp.full((_R, pdim), -1e30, jnp.float32)

    def agg_body(ci, carry):
        macc, xacc = carry
        cg = ci0 + ci
        d2, same = chunk_cols(cg)
        sel = same & (d2 <= tau)
        w = jnp.exp(-10.0 * jnp.maximum(d2, 0.0))
        wm = jnp.where(sel, w, 0.0)
        penal = jnp.where(sel, 0.0, -1e30)
        h_c = h3_ref[cg]
        macc = macc + jnp.dot(wm, h_c, preferred_element_type=jnp.float32)
        ht = ht3_ref[cg]
        cols = []
        for p in range(pdim):
            cols.append(jnp.max(wm * ht[p:p + 1, :] + penal,
                                axis=1, keepdims=True))
        xacc = jnp.maximum(xacc, jnp.concatenate(cols, axis=1))
        return macc, xacc

    macc, xacc = lax.fori_loop(0, nc, agg_body, (macc0, xacc0))
    xacc = jnp.where(xacc < -1e20, 0.0, xacc)
    feat = jnp.concatenate([macc * (1.0 / _K), xacc], axis=1)

    out = out1_ref[...] + jnp.dot(feat, wo2_ref[...],
                                  preferred_element_type=jnp.float32) + bo2_ref[...]
    p1 = jnp.tanh(jnp.dot(out, wp1_ref[...],
                          preferred_element_type=jnp.float32) + bp1_ref[...])
    pv = jnp.tanh(jnp.dot(p1, wp2_ref[...],
                          preferred_element_type=jnp.float32) + bp2_ref[...])
    p_ref[...] = pv

    @pl.when(t == 0)
    def _():
        ssum_ref[...] = jnp.zeros_like(ssum_ref)
        smin_ref[...] = jnp.full_like(smin_ref, jnp.inf)
        smax_ref[...] = jnp.full_like(smax_ref, -jnp.inf)
        scnt_ref[...] = jnp.zeros_like(scnt_ref)

    for b in range(_B):
        maskb = brow == b
        ssum_ref[b:b + 1, :] += jnp.sum(jnp.where(maskb, pv, 0.0),
                                        axis=0, keepdims=True)
        smin_ref[b:b + 1, :] = jnp.minimum(
            smin_ref[b:b + 1, :],
            jnp.min(jnp.where(maskb, pv, jnp.inf), axis=0, keepdims=True))
        smax_ref[b:b + 1, :] = jnp.maximum(
            smax_ref[b:b + 1, :],
            jnp.max(jnp.where(maskb, pv, -jnp.inf), axis=0, keepdims=True))
        scnt_ref[b:b + 1, :] += jnp.sum(
            jnp.where(maskb, 1.0, 0.0), axis=0, keepdims=True)


def _head_kernel(p_ref, brow_ref, ssum_ref, smin_ref, smax_ref, scnt_ref,
                 wa_ref, wb_ref, bo_ref, y_ref):
    cnt = scnt_ref[:, 0:1]
    pos = cnt > 0.0
    mean = jnp.where(pos, ssum_ref[...] / cnt, 0.0)
    mn = jnp.where(pos, smin_ref[...], 0.0)
    mx = jnp.where(pos, smax_ref[...], 0.0)
    mmm = jnp.concatenate([mean, mn, mx], axis=1)
    brow = brow_ref[...]
    seg = lax.broadcasted_iota(jnp.int32, (1, _B), 1)
    onehot = (brow == seg).astype(jnp.float32)
    gxa = jnp.dot(onehot, mmm, preferred_element_type=jnp.float32)
    y_ref[...] = jnp.tanh(
        jnp.dot(gxa, wa_ref[...], preferred_element_type=jnp.float32)
        + jnp.dot(p_ref[...], wb_ref[...], preferred_element_type=jnp.float32)
        + bo_ref[...])


def kernel(g, x, batch, W_s, b_s, W_h, b_h, W_o1, W_o2, b_o2,
           W_p1, b_p1, W_p2, b_p2, W_out, b_out):
    n, din = x.shape
    sdim = W_s.shape[1]
    pdim = W_h.shape[1]
    dout = W_o1.shape[1]
    dh = W_p2.shape[1]
    nch = _cdiv(n, _C)
    npad = nch * _C
    nt = _cdiv(n, _R)
    assert nt * _R == npad

    wcat = jnp.concatenate([W_s, W_h, W_o1], axis=1)
    bcat = jnp.concatenate(
        [b_s, b_h, jnp.zeros((dout,), jnp.float32)]).reshape(1, -1)
    dcat = wcat.shape[1]
    xp = jnp.pad(x, ((0, npad - n), (0, 0)))
    cat, sq = pl.pallas_call(
        partial(_embed_kernel, sdim=sdim),
        grid=(npad // 512,),
        in_specs=[pl.BlockSpec((512, din), lambda i: (i, 0)),
                  pl.BlockSpec((din, dcat), lambda i: (0, 0)),
                  pl.BlockSpec((1, dcat), lambda i: (0, 0))],
        out_specs=[pl.BlockSpec((512, dcat), lambda i: (i, 0)),
                   pl.BlockSpec((512, 1), lambda i: (i, 0))],
        out_shape=[jax.ShapeDtypeStruct((npad, dcat), jnp.float32),
                   jax.ShapeDtypeStruct((npad, 1), jnp.float32)],
    )(xp, wcat, bcat)
    s = cat[:, :sdim]
    h = cat[:, sdim:sdim + pdim]
    out1 = cat[:, sdim + pdim:]

    batch = batch.astype(jnp.int32)
    starts = jnp.searchsorted(
        batch, jnp.arange(_B + 1, dtype=jnp.int32)).astype(jnp.int32)
    tstart = jnp.minimum(jnp.arange(nt, dtype=jnp.int32) * _R, n - 1)
    tend = jnp.minimum(tstart + _R - 1, n - 1)
    c0 = starts[batch[tstart]]
    c1 = starts[batch[tend] + 1]
    ci0 = c0 // _C
    nc = jnp.maximum(_cdiv(c1, _C) - ci0, 1)
    c01 = jnp.stack([ci0, nc], axis=1)

    batchp = jnp.concatenate(
        [batch, jnp.full((npad - n,), -1, jnp.int32)])
    s3 = s.T.reshape(sdim, nch, _C).transpose(1, 0, 2)
    sq3 = sq.reshape(nch, _C)[:, None, :]
    b3 = batchp.reshape(nch, _C)[:, None, :]
    h3 = h.reshape(nch, _C, pdim)
    ht3 = h.T.reshape(pdim, nch, _C).transpose(1, 0, 2)
    brows = batchp.reshape(npad, 1)

    full3 = lambda t, c: (0, 0, 0)
    full2 = lambda t, c: (0, 0)
    rowblk = lambda t, c: (t, 0)
    grid_spec = pltpu.PrefetchScalarGridSpec(
        num_scalar_prefetch=1,
        grid=(nt,),
        in_specs=[
            pl.BlockSpec((_R, sdim), rowblk),
            pl.BlockSpec((_R, 1), rowblk),
            pl.BlockSpec((_R, dout), rowblk),
            pl.BlockSpec((_R, 1), rowblk),
            pl.BlockSpec((nch, sdim, _C), full3),
            pl.BlockSpec((nch, 1, _C), full3),
            pl.BlockSpec((nch, 1, _C), full3),
            pl.BlockSpec((nch, _C, pdim), full3),
            pl.BlockSpec((nch, pdim, _C), full3),
            pl.BlockSpec(W_o2.shape, full2),
            pl.BlockSpec((1, dout), full2),
            pl.BlockSpec(W_p1.shape, full2),
            pl.BlockSpec((1, W_p1.shape[1]), full2),
            pl.BlockSpec(W_p2.shape, full2),
            pl.BlockSpec((1, dh), full2),
        ],
        out_specs=[
            pl.BlockSpec((_R, _K), rowblk),
            pl.BlockSpec((_R, dh), rowblk),
            pl.BlockSpec((_B, dh), full2),
            pl.BlockSpec((_B, dh), full2),
            pl.BlockSpec((_B, dh), full2),
            pl.BlockSpec((_B, 128), full2),
        ],
        scratch_shapes=[pltpu.VMEM((nch, _R, _C), jnp.float32)],
    )
    idx, pvals, ssum, smin, smax, scnt = pl.pallas_call(
        partial(_knn_kernel, pdim=pdim),
        grid_spec=grid_spec,
        out_shape=[jax.ShapeDtypeStruct((npad, _K), jnp.int32),
                   jax.ShapeDtypeStruct((npad, dh), jnp.float32),
                   jax.ShapeDtypeStruct((_B, dh), jnp.float32),
                   jax.ShapeDtypeStruct((_B, dh), jnp.float32),
                   jax.ShapeDtypeStruct((_B, dh), jnp.float32),
                   jax.ShapeDtypeStruct((_B, 128), jnp.float32)],
        compiler_params=pltpu.CompilerParams(
            dimension_semantics=("arbitrary",)),
    )(c01, s, sq, out1, brows, s3, sq3, b3, h3, ht3,
      W_o2, b_o2.reshape(1, -1), W_p1, b_p1.reshape(1, -1),
      W_p2, b_p2.reshape(1, -1))

    wa = W_out[:3 * dh]
    wb = W_out[3 * dh:]
    y = pl.pallas_call(
        _head_kernel,
        grid=(nt,),
        in_specs=[pl.BlockSpec((_R, dh), lambda t: (t, 0)),
                  pl.BlockSpec((_R, 1), lambda t: (t, 0)),
                  pl.BlockSpec((_B, dh), lambda t: (0, 0)),
                  pl.BlockSpec((_B, dh), lambda t: (0, 0)),
                  pl.BlockSpec((_B, dh), lambda t: (0, 0)),
                  pl.BlockSpec((_B, 128), lambda t: (0, 0)),
                  pl.BlockSpec(wa.shape, lambda t: (0, 0)),
                  pl.BlockSpec(wb.shape, lambda t: (0, 0)),
                  pl.BlockSpec((1, W_out.shape[1]), lambda t: (0, 0))],
        out_specs=pl.BlockSpec((_R, W_out.shape[1]), lambda t: (t, 0)),
        out_shape=jax.ShapeDtypeStruct((npad, W_out.shape[1]), jnp.float32),
    )(pvals, brows, ssum, smin, smax, scnt, wa, wb, b_out.reshape(1, -1))

    row1 = jnp.repeat(jnp.arange(n, dtype=jnp.int32), _K)
    edge_index = jnp.stack([idx[:n].reshape(-1), row1], axis=0)
    return y[:n], edge_index


# speedup vs baseline: 1.4244x; 1.4244x over previous
from functools import partial

import jax
import jax.numpy as jnp
from jax import lax
from jax.experimental import pallas as pl
from jax.experimental.pallas import tpu as pltpu

_R = 512
_C = 1024
_K = 40
_B = 8
_NEG = float("-inf")


def _cdiv(a, b):
    return (a + b - 1) // b


def _embed_kernel(x_ref, w_ref, b_ref, cat_ref, sq_ref, *, sdim):
    cat = jnp.dot(x_ref[...], w_ref[...],
                  preferred_element_type=jnp.float32) + b_ref[...]
    cat_ref[...] = cat
    s = cat[:, :sdim]
    sq_ref[...] = jnp.sum(s * s, axis=1, keepdims=True)


def _knn_kernel(c01_ref, s_rows_ref, sq_rows_ref, out1_ref, brow_ref,
                s3_ref, sq3_ref, b3_ref, h3_ref, ht3_ref,
                wo2_ref, bo2_ref, wp1_ref, bp1_ref, wp2_ref, bp2_ref,
                idx_ref, p_ref, ssum_ref, smin_ref, smax_ref, scnt_ref,
                negbuf, *, pdim):
    t = pl.program_id(0)
    ci0 = c01_ref[t, 0]
    nc = c01_ref[t, 1]

    s_rows = s_rows_ref[...]
    sqr = sq_rows_ref[...]
    brow = brow_ref[...]
    lane = lax.broadcasted_iota(jnp.int32, (_R, _C), 1)

    def chunk_cols(cg):
        s_ct = s3_ref[cg]
        sq_c = sq3_ref[cg]
        b_c = b3_ref[cg]
        dot = lax.dot_general(s_rows, s_ct, (((1,), (0,)), ((), ())),
                              preferred_element_type=jnp.float32)
        d2 = (sqr + sq_c) - 2.0 * dot
        same = brow == b_c
        return d2, same

    def fill_body(ci, _):
        d2, same = chunk_cols(ci0 + ci)
        negbuf[ci] = jnp.where(same, -d2, _NEG)
        return 0
    lax.fori_loop(0, nc, fill_body, 0)

    big_i = jnp.int32(2 ** 30)

    def k_body(k, carry):
        topi, ntau, pj = carry

        def scan_body(ci, bc):
            bv, bj = bc
            off = (ci0 + ci) * _C
            chunk = jnp.where(lane == pj - off, _NEG, negbuf[ci])
            negbuf[ci] = chunk
            m = jnp.max(chunk, axis=1, keepdims=True)
            cand = jnp.where(chunk == m, lane, big_i)
            j = jnp.min(cand, axis=1, keepdims=True) + off
            upd = m > bv
            return jnp.where(upd, m, bv), jnp.where(upd, j, bj)

        bv0 = jnp.full((_R, 1), _NEG, jnp.float32)
        bj0 = jnp.zeros((_R, 1), jnp.int32)
        bv, bj = lax.fori_loop(0, nc, scan_body, (bv0, bj0))

        kl = lax.broadcasted_iota(jnp.int32, (1, _K), 1)
        topi = jnp.where(kl == k, bj, topi)
        ntau = jnp.where(k == _K - 1, bv, ntau)
        return topi, ntau, bj

    topi0 = jnp.zeros((_R, _K), jnp.int32)
    ntau0 = jnp.zeros((_R, 1), jnp.float32)
    pj0 = jnp.full((_R, 1), -1, jnp.int32)
    topi, ntau, _ = lax.fori_loop(0, _K, k_body, (topi0, ntau0, pj0))
    idx_ref[...] = topi
    tau = -ntau

    macc0 = jnp.zeros((_R, pdim), jnp.float32)
    xacc0 = jnp.full((_R, pdim), -1e30, jnp.float32)

    def agg_body(ci, carry):
        macc, xacc = carry
        cg = ci0 + ci
        d2, same = chunk_cols(cg)
        sel = same & (d2 <= tau)
        w = jnp.exp(-10.0 * jnp.maximum(d2, 0.0))
        wm = jnp.where(sel, w, 0.0)
        penal = jnp.where(sel, 0.0, -1e30)
        h_c = h3_ref[cg]
        macc = macc + jnp.dot(wm, h_c, preferred_element_type=jnp.float32)
        ht = ht3_ref[cg]
        cols = []
        for p in range(pdim):
            cols.append(jnp.max(wm * ht[p:p + 1, :] + penal,
                                axis=1, keepdims=True))
        xacc = jnp.maximum(xacc, jnp.concatenate(cols, axis=1))
        return macc, xacc

    macc, xacc = lax.fori_loop(0, nc, agg_body, (macc0, xacc0))
    xacc = jnp.where(xacc < -1e20, 0.0, xacc)
    feat = jnp.concatenate([macc * (1.0 / _K), xacc], axis=1)

    out = out1_ref[...] + jnp.dot(feat, wo2_ref[...],
                                  preferred_element_type=jnp.float32) + bo2_ref[...]
    p1 = jnp.tanh(jnp.dot(out, wp1_ref[...],
                          preferred_element_type=jnp.float32) + bp1_ref[...])
    pv = jnp.tanh(jnp.dot(p1, wp2_ref[...],
                          preferred_element_type=jnp.float32) + bp2_ref[...])
    p_ref[...] = pv

    @pl.when(t == 0)
    def _():
        ssum_ref[...] = jnp.zeros_like(ssum_ref)
        smin_ref[...] = jnp.full_like(smin_ref, jnp.inf)
        smax_ref[...] = jnp.full_like(smax_ref, -jnp.inf)
        scnt_ref[...] = jnp.zeros_like(scnt_ref)

    for b in range(_B):
        maskb = brow == b
        ssum_ref[b:b + 1, :] += jnp.sum(jnp.where(maskb, pv, 0.0),
                                        axis=0, keepdims=True)
        smin_ref[b:b + 1, :] = jnp.minimum(
            smin_ref[b:b + 1, :],
            jnp.min(jnp.where(maskb, pv, jnp.inf), axis=0, keepdims=True))
        smax_ref[b:b + 1, :] = jnp.maximum(
            smax_ref[b:b + 1, :],
            jnp.max(jnp.where(maskb, pv, -jnp.inf), axis=0, keepdims=True))
        scnt_ref[b:b + 1, :] += jnp.sum(
            jnp.where(maskb, 1.0, 0.0), axis=0, keepdims=True)


def _head_kernel(p_ref, brow_ref, ssum_ref, smin_ref, smax_ref, scnt_ref,
                 wa_ref, wb_ref, bo_ref, y_ref):
    cnt = scnt_ref[:, 0:1]
    pos = cnt > 0.0
    mean = jnp.where(pos, ssum_ref[...] / cnt, 0.0)
    mn = jnp.where(pos, smin_ref[...], 0.0)
    mx = jnp.where(pos, smax_ref[...], 0.0)
    mmm = jnp.concatenate([mean, mn, mx], axis=1)
    brow = brow_ref[...]
    seg = lax.broadcasted_iota(jnp.int32, (1, _B), 1)
    onehot = (brow == seg).astype(jnp.float32)
    gxa = jnp.dot(onehot, mmm, preferred_element_type=jnp.float32)
    y_ref[...] = jnp.tanh(
        jnp.dot(gxa, wa_ref[...], preferred_element_type=jnp.float32)
        + jnp.dot(p_ref[...], wb_ref[...], preferred_element_type=jnp.float32)
        + bo_ref[...])


def kernel(g, x, batch, W_s, b_s, W_h, b_h, W_o1, W_o2, b_o2,
           W_p1, b_p1, W_p2, b_p2, W_out, b_out):
    n, din = x.shape
    sdim = W_s.shape[1]
    pdim = W_h.shape[1]
    dout = W_o1.shape[1]
    dh = W_p2.shape[1]
    nch = _cdiv(n, _C)
    npad = nch * _C
    nt = _cdiv(n, _R)
    assert nt * _R == npad

    wcat = jnp.concatenate([W_s, W_h, W_o1], axis=1)
    bcat = jnp.concatenate(
        [b_s, b_h, jnp.zeros((dout,), jnp.float32)]).reshape(1, -1)
    dcat = wcat.shape[1]
    xp = jnp.pad(x, ((0, npad - n), (0, 0)))
    cat, sq = pl.pallas_call(
        partial(_embed_kernel, sdim=sdim),
        grid=(npad // 512,),
        in_specs=[pl.BlockSpec((512, din), lambda i: (i, 0)),
                  pl.BlockSpec((din, dcat), lambda i: (0, 0)),
                  pl.BlockSpec((1, dcat), lambda i: (0, 0))],
        out_specs=[pl.BlockSpec((512, dcat), lambda i: (i, 0)),
                   pl.BlockSpec((512, 1), lambda i: (i, 0))],
        out_shape=[jax.ShapeDtypeStruct((npad, dcat), jnp.float32),
                   jax.ShapeDtypeStruct((npad, 1), jnp.float32)],
    )(xp, wcat, bcat)
    s = cat[:, :sdim]
    h = cat[:, sdim:sdim + pdim]
    out1 = cat[:, sdim + pdim:]

    batch = batch.astype(jnp.int32)
    starts = jnp.searchsorted(
        batch, jnp.arange(_B + 1, dtype=jnp.int32)).astype(jnp.int32)
    tstart = jnp.minimum(jnp.arange(nt, dtype=jnp.int32) * _R, n - 1)
    tend = jnp.minimum(tstart + _R - 1, n - 1)
    c0 = starts[batch[tstart]]
    c1 = starts[batch[tend] + 1]
    ci0 = c0 // _C
    nc = jnp.maximum(_cdiv(c1, _C) - ci0, 1)
    c01 = jnp.stack([ci0, nc], axis=1)

    batchp = jnp.concatenate(
        [batch, jnp.full((npad - n,), -1, jnp.int32)])
    s3 = s.T.reshape(sdim, nch, _C).transpose(1, 0, 2)
    sq3 = sq.reshape(nch, _C)[:, None, :]
    b3 = batchp.reshape(nch, _C)[:, None, :]
    h3 = h.reshape(nch, _C, pdim)
    ht3 = h.T.reshape(pdim, nch, _C).transpose(1, 0, 2)
    brows = batchp.reshape(npad, 1)

    full3 = lambda t, c: (0, 0, 0)
    full2 = lambda t, c: (0, 0)
    rowblk = lambda t, c: (t, 0)
    grid_spec = pltpu.PrefetchScalarGridSpec(
        num_scalar_prefetch=1,
        grid=(nt,),
        in_specs=[
            pl.BlockSpec((_R, sdim), rowblk),
            pl.BlockSpec((_R, 1), rowblk),
            pl.BlockSpec((_R, dout), rowblk),
            pl.BlockSpec((_R, 1), rowblk),
            pl.BlockSpec((nch, sdim, _C), full3),
            pl.BlockSpec((nch, 1, _C), full3),
            pl.BlockSpec((nch, 1, _C), full3),
            pl.BlockSpec((nch, _C, pdim), full3),
            pl.BlockSpec((nch, pdim, _C), full3),
            pl.BlockSpec(W_o2.shape, full2),
            pl.BlockSpec((1, dout), full2),
            pl.BlockSpec(W_p1.shape, full2),
            pl.BlockSpec((1, W_p1.shape[1]), full2),
            pl.BlockSpec(W_p2.shape, full2),
            pl.BlockSpec((1, dh), full2),
        ],
        out_specs=[
            pl.BlockSpec((_R, _K), rowblk),
            pl.BlockSpec((_R, dh), rowblk),
            pl.BlockSpec((_B, dh), full2),
            pl.BlockSpec((_B, dh), full2),
            pl.BlockSpec((_B, dh), full2),
            pl.BlockSpec((_B, 128), full2),
        ],
        scratch_shapes=[pltpu.VMEM((nch, _R, _C), jnp.float32)],
    )
    idx, pvals, ssum, smin, smax, scnt = pl.pallas_call(
        partial(_knn_kernel, pdim=pdim),
        grid_spec=grid_spec,
        out_shape=[jax.ShapeDtypeStruct((npad, _K), jnp.int32),
                   jax.ShapeDtypeStruct((npad, dh), jnp.float32),
                   jax.ShapeDtypeStruct((_B, dh), jnp.float32),
                   jax.ShapeDtypeStruct((_B, dh), jnp.float32),
                   jax.ShapeDtypeStruct((_B, dh), jnp.float32),
                   jax.ShapeDtypeStruct((_B, 128), jnp.float32)],
        compiler_params=pltpu.CompilerParams(
            dimension_semantics=("arbitrary",)),
    )(c01, s, sq, out1, brows, s3, sq3, b3, h3, ht3,
      W_o2, b_o2.reshape(1, -1), W_p1, b_p1.reshape(1, -1),
      W_p2, b_p2.reshape(1, -1))

    wa = W_out[:3 * dh]
    wb = W_out[3 * dh:]
    y = pl.pallas_call(
        _head_kernel,
        grid=(nt,),
        in_specs=[pl.BlockSpec((_R, dh), lambda t: (t, 0)),
                  pl.BlockSpec((_R, 1), lambda t: (t, 0)),
                  pl.BlockSpec((_B, dh), lambda t: (0, 0)),
                  pl.BlockSpec((_B, dh), lambda t: (0, 0)),
                  pl.BlockSpec((_B, dh), lambda t: (0, 0)),
                  pl.BlockSpec((_B, 128), lambda t: (0, 0)),
                  pl.BlockSpec(wa.shape, lambda t: (0, 0)),
                  pl.BlockSpec(wb.shape, lambda t: (0, 0)),
                  pl.BlockSpec((1, W_out.shape[1]), lambda t: (0, 0))],
        out_specs=pl.BlockSpec((_R, W_out.shape[1]), lambda t: (t, 0)),
        out_shape=jax.ShapeDtypeStruct((npad, W_out.shape[1]), jnp.float32),
    )(pvals, brows, ssum, smin, smax, scnt, wa, wb, b_out.reshape(1, -1))

    row1 = jnp.repeat(jnp.arange(n, dtype=jnp.int32), _K)
    edge_index = jnp.stack([idx[:n].reshape(-1), row1], axis=0)
    return y[:n], edge_index
